# baseline (speedup 1.0000x reference)
import functools

import jax
import jax.numpy as jnp
from jax import lax
from jax.experimental import pallas as pl
from jax.experimental.pallas import tpu as pltpu
from jax.experimental.pallas import tpu_sc as plsc

N = 10000
E = 320000
D_IN = 128
D_EDGE = 16
C = 64
G = 64

NC = 2
NS = 16
NW = NC * NS
EPW = E // NW
K = 80
NB = EPW // K
NPAD = 10112
NPT = NPAD // NS
ZROWS = 8
ROW = 80

EBLK = E // 16


def _ea_sum_body(ea_ref, out_ref):
    i = pl.program_id(0)

    @pl.when(i == 0)
    def _():
        out_ref[...] = jnp.zeros_like(out_ref)

    out_ref[...] += jnp.sum(ea_ref[...], axis=0, keepdims=True)


def _ea_colsum(edge_attr):
    return pl.pallas_call(
        _ea_sum_body,
        grid=(16,),
        in_specs=[pl.BlockSpec((EBLK, D_EDGE), lambda i: (i, 0))],
        out_specs=pl.BlockSpec((1, D_EDGE), lambda i: (0, 0)),
        out_shape=jax.ShapeDtypeStruct((1, D_EDGE), jnp.float32),
    )(edge_attr)


def _el_body(ea_ref, we_ref, ae_ref, el_ref):
    t = jnp.dot(ea_ref[...], we_ref[...], preferred_element_type=jnp.float32)
    el_ref[...] = jnp.dot(t, ae_ref[...], preferred_element_type=jnp.float32)


def _edge_logits(edge_attr, We, ae):
    return pl.pallas_call(
        _el_body,
        grid=(16,),
        in_specs=[
            pl.BlockSpec((EBLK, D_EDGE), lambda i: (i, 0)),
            pl.BlockSpec((D_EDGE, C), lambda i: (0, 0)),
            pl.BlockSpec((C, 1), lambda i: (0, 0)),
        ],
        out_specs=pl.BlockSpec((EBLK, 1), lambda i: (i, 0)),
        out_shape=jax.ShapeDtypeStruct((E, 1), jnp.float32),
    )(edge_attr, We, ae)


def _prep_body(x_ref, w_ref, av_ref, we_ref, ae_ref, es_ref,
               xl_ref, sd_ref, as_ref):
    xl = jnp.dot(x_ref[...], w_ref[...], preferred_element_type=jnp.float32)
    xl_ref[...] = xl
    sd = jnp.dot(xl, av_ref[...], preferred_element_type=jnp.float32)
    sd_ref[...] = sd
    t = jnp.dot(es_ref[...] * (1.0 / E), we_ref[...],
                preferred_element_type=jnp.float32)
    c0 = jnp.dot(t, ae_ref[...], preferred_element_type=jnp.float32)
    a = sd[:, 0:1] + sd[:, 1:2] + c0
    a = jnp.where(a >= 0, a, 0.2 * a)
    as_ref[...] = jnp.exp(a)


def _prep(xin, W, avec, We, ae, ea_sum):
    n, _ = xin.shape
    return pl.pallas_call(
        _prep_body,
        out_shape=(
            jax.ShapeDtypeStruct((n, C), jnp.float32),
            jax.ShapeDtypeStruct((n, 2), jnp.float32),
            jax.ShapeDtypeStruct((n, 1), jnp.float32),
        ),
    )(xin, W, avec, We, ae, ea_sum)


def _comb_body(acc_ref, as_ref, xl_ref, b_ref, h_ref):
    ns = acc_ref[0:N, :] + acc_ref[NPAD:NPAD + N, :]
    num = ns[:, 0:C] + as_ref[...] * xl_ref[...]
    den = ns[:, C:C + 1] + as_ref[...] + 1e-16
    h_ref[...] = num / den + b_ref[...]


def _combine(acc, aself, xl, brow):
    return pl.pallas_call(
        _comb_body,
        out_shape=jax.ShapeDtypeStruct((N, C), jnp.float32),
    )(acc, aself, xl, brow)


def _pool_body(acc_ref, as_ref, xl_ref, b_ref, batch_ref, out_ref):
    ns = acc_ref[0:N, :] + acc_ref[NPAD:NPAD + N, :]
    num = ns[:, 0:C] + as_ref[...] * xl_ref[...]
    den = ns[:, C:C + 1] + as_ref[...] + 1e-16
    h = num / den + b_ref[...]
    gids = lax.broadcasted_iota(jnp.int32, (G, N), 0)
    p = (gids == batch_ref[...]).astype(jnp.float32)
    out_ref[...] = jnp.dot(p, h, preferred_element_type=jnp.float32,
                           precision=lax.Precision.HIGHEST)


def _combine_pool(acc, aself, xl, brow, batch_row):
    return pl.pallas_call(
        _pool_body,
        out_shape=jax.ShapeDtypeStruct((G, C), jnp.float32),
    )(acc, aself, xl, brow, batch_row)


def _sc_body(xl_hbm, s_hbm, d_hbm, src_hbm, dst_hbm, el_hbm, out_hbm,
             s_v, d_v, srcb, dstb, elb, rows, srows, exb, zb, acc, gsem):
    cid = lax.axis_index("c")
    sid = lax.axis_index("s")
    wid = cid * NS + sid

    pltpu.sync_copy(s_hbm, s_v)
    pltpu.sync_copy(d_hbm, d_v)
    pltpu.sync_copy(src_hbm.at[wid], srcb)
    pltpu.sync_copy(dst_hbm.at[wid], dstb)
    pltpu.sync_copy(el_hbm.at[pl.ds(wid * EPW, EPW)], elb)

    def zrow(r, carry):
        for cc in range(ROW // 16):
            zb[r, pl.ds(cc * 16, 16)] = jnp.zeros((16,), jnp.float32)
        return carry

    lax.fori_loop(0, ZROWS, zrow, 0)
    for t in range(NPT // ZROWS):
        pltpu.sync_copy(zb, acc.at[pl.ds(sid * NPT + t * ZROWS, ZROWS)])
    plsc.subcore_barrier()

    iota16 = lax.iota(jnp.int32, 16)

    def phase(b, carry):
        cp = pltpu.async_copy(xl_hbm.at[srcb.at[b]], rows, gsem)
        for gi in range(K // 16):
            srci = srcb[b, pl.ds(gi * 16, 16)]
            dsti = dstb[b, pl.ds(gi * 16, 16)]
            sv = plsc.load_gather(s_v, [srci])
            dv = plsc.load_gather(d_v, [dsti])
            a = sv + dv + elb[pl.ds(b * K + gi * 16, 16)]
            a = jnp.where(a >= 0, a, 0.2 * a)
            exb[pl.ds(gi * 16, 16)] = jnp.exp(a)
        cp.wait()

        def row_body(j, rcarry):
            exj = plsc.load_gather(exb, [jnp.full((16,), j, jnp.int32)])
            for cc in range(C // 16):
                srows[j, pl.ds(cc * 16, 16)] = rows[j, pl.ds(cc * 16, 16)] * exj
            srows[j, pl.ds(C, 16)] = jnp.where(iota16 == 0, exj, 0.0)
            return rcarry

        lax.fori_loop(0, K, row_body, 0)
        pltpu.sync_copy(srows, acc.at[dstb.at[b]], add=True)
        return carry

    lax.fori_loop(0, NB, phase, 0)

    plsc.subcore_barrier()
    for t in range(NPT // ZROWS):
        off = sid * NPT + t * ZROWS
        pltpu.sync_copy(acc.at[pl.ds(off, ZROWS)],
                        out_hbm.at[pl.ds(cid * NPAD + off, ZROWS)])


@functools.cache
def _make_sc_edge_pass():
    return pl.kernel(
        _sc_body,
        out_type=jax.ShapeDtypeStruct((NC * NPAD, ROW), jnp.float32),
        compiler_params=pltpu.CompilerParams(
            needs_layout_passes=False, use_tc_tiling_on_sc=False),
        mesh=plsc.VectorSubcoreMesh(
            core_axis_name="c", subcore_axis_name="s",
            num_cores=NC, num_subcores=NS),
        scratch_types=[
            pltpu.VMEM((N,), jnp.float32),
            pltpu.VMEM((N,), jnp.float32),
            pltpu.VMEM((NB, K), jnp.int32),
            pltpu.VMEM((NB, K), jnp.int32),
            pltpu.VMEM((EPW,), jnp.float32),
            pltpu.VMEM((K, C), jnp.float32),
            pltpu.VMEM((K, ROW), jnp.float32),
            pltpu.VMEM((K,), jnp.float32),
            pltpu.VMEM((ZROWS, ROW), jnp.float32),
            pltpu.MemorySpace.VMEM_SHARED((NPAD, ROW), jnp.float32),
            pltpu.SemaphoreType.DMA,
        ],
    )


def kernel(x, edge_index, edge_attr, batch, W1, a_src1, a_dst1, We1, a_e1, b1,
           W2, a_src2, a_dst2, We2, a_e2, b2):
    src3 = edge_index[0].reshape(NW, NB, K)
    dst3 = edge_index[1].reshape(NW, NB, K)
    batch_row = batch.reshape(1, N)

    ea_sum = _ea_colsum(edge_attr)

    avec1 = jnp.stack([a_src1, a_dst1], axis=1)
    avec2 = jnp.stack([a_src2, a_dst2], axis=1)

    el1 = _edge_logits(edge_attr, We1, a_e1.reshape(C, 1)).reshape(E)
    el2 = _edge_logits(edge_attr, We2, a_e2.reshape(C, 1)).reshape(E)

    sc_pass = _make_sc_edge_pass()

    xl1, sd1, aself1 = _prep(x, W1, avec1, We1, a_e1.reshape(C, 1), ea_sum)
    acc1 = sc_pass(xl1, sd1[:, 0], sd1[:, 1], src3, dst3, el1)
    h1 = _combine(acc1, aself1, xl1, b1.reshape(1, C))

    xl2, sd2, aself2 = _prep(h1, W2, avec2, We2, a_e2.reshape(C, 1), ea_sum)
    acc2 = sc_pass(xl2, sd2[:, 0], sd2[:, 1], src3, dst3, el2)
    return _combine_pool(acc2, aself2, xl2, b2.reshape(1, C), batch_row)


# speedup vs baseline: 1.0842x; 1.0842x over previous
import functools

import jax
import jax.numpy as jnp
from jax import lax
from jax.experimental import pallas as pl
from jax.experimental.pallas import tpu as pltpu
from jax.experimental.pallas import tpu_sc as plsc

N = 10000
E = 320000
D_IN = 128
D_EDGE = 16
C = 64
G = 64

NC = 2
NS = 16
NW = NC * NS
EPW = E // NW
K = 80
NB = EPW // K
NPAD = 10240
NPT = NPAD // NS
ZROWS = 128
ROW = 80

EBLK = E // 16


def _ea_sum_body(ea_ref, out_ref):
    i = pl.program_id(0)

    @pl.when(i == 0)
    def _():
        out_ref[...] = jnp.zeros_like(out_ref)

    out_ref[...] += jnp.sum(ea_ref[...], axis=0, keepdims=True)


def _ea_colsum(edge_attr):
    return pl.pallas_call(
        _ea_sum_body,
        grid=(16,),
        in_specs=[pl.BlockSpec((EBLK, D_EDGE), lambda i: (i, 0))],
        out_specs=pl.BlockSpec((1, D_EDGE), lambda i: (0, 0)),
        out_shape=jax.ShapeDtypeStruct((1, D_EDGE), jnp.float32),
    )(edge_attr)


def _el_body(ea_ref, we_ref, ae_ref, el_ref):
    t = jnp.dot(ea_ref[...], we_ref[...], preferred_element_type=jnp.float32)
    el_ref[...] = jnp.dot(t, ae_ref[...], preferred_element_type=jnp.float32)


def _edge_logits(edge_attr, We, ae):
    return pl.pallas_call(
        _el_body,
        grid=(16,),
        in_specs=[
            pl.BlockSpec((EBLK, D_EDGE), lambda i: (i, 0)),
            pl.BlockSpec((D_EDGE, C), lambda i: (0, 0)),
            pl.BlockSpec((C, 1), lambda i: (0, 0)),
        ],
        out_specs=pl.BlockSpec((EBLK, 1), lambda i: (i, 0)),
        out_shape=jax.ShapeDtypeStruct((E, 1), jnp.float32),
    )(edge_attr, We, ae)


def _prep_body(x_ref, w_ref, av_ref, we_ref, ae_ref, es_ref,
               xl_ref, sd_ref, as_ref):
    xl = jnp.dot(x_ref[...], w_ref[...], preferred_element_type=jnp.float32)
    xl_ref[...] = xl
    sd = jnp.dot(xl, av_ref[...], preferred_element_type=jnp.float32)
    sd_ref[...] = sd
    t = jnp.dot(es_ref[...] * (1.0 / E), we_ref[...],
                preferred_element_type=jnp.float32)
    c0 = jnp.dot(t, ae_ref[...], preferred_element_type=jnp.float32)
    a = sd[:, 0:1] + sd[:, 1:2] + c0
    a = jnp.where(a >= 0, a, 0.2 * a)
    as_ref[...] = jnp.exp(a)


def _prep(xin, W, avec, We, ae, ea_sum):
    n, _ = xin.shape
    return pl.pallas_call(
        _prep_body,
        out_shape=(
            jax.ShapeDtypeStruct((n, C), jnp.float32),
            jax.ShapeDtypeStruct((n, 2), jnp.float32),
            jax.ShapeDtypeStruct((n, 1), jnp.float32),
        ),
    )(xin, W, avec, We, ae, ea_sum)


def _comb_body(acc_ref, as_ref, xl_ref, b_ref, h_ref):
    ns = acc_ref[0:N, :] + acc_ref[NPAD:NPAD + N, :]
    num = ns[:, 0:C] + as_ref[...] * xl_ref[...]
    den = ns[:, C:C + 1] + as_ref[...] + 1e-16
    h_ref[...] = num / den + b_ref[...]


def _combine(acc, aself, xl, brow):
    return pl.pallas_call(
        _comb_body,
        out_shape=jax.ShapeDtypeStruct((N, C), jnp.float32),
    )(acc, aself, xl, brow)


def _pool_body(acc_ref, as_ref, xl_ref, b_ref, batch_ref, out_ref):
    ns = acc_ref[0:N, :] + acc_ref[NPAD:NPAD + N, :]
    num = ns[:, 0:C] + as_ref[...] * xl_ref[...]
    den = ns[:, C:C + 1] + as_ref[...] + 1e-16
    h = num / den + b_ref[...]
    gids = lax.broadcasted_iota(jnp.int32, (G, N), 0)
    p = (gids == batch_ref[...]).astype(jnp.float32)
    out_ref[...] = jnp.dot(p, h, preferred_element_type=jnp.float32,
                           precision=lax.Precision.HIGHEST)


def _combine_pool(acc, aself, xl, brow, batch_row):
    return pl.pallas_call(
        _pool_body,
        out_shape=jax.ShapeDtypeStruct((G, C), jnp.float32),
    )(acc, aself, xl, brow, batch_row)


def _sc_body(xl_hbm, s_hbm, d_hbm, src_hbm, dst_hbm, el_hbm, out_hbm,
             s_v, d_v, srcb, dstb, elb, rows, srows, exb, zb, acc, gsem):
    cid = lax.axis_index("c")
    sid = lax.axis_index("s")
    wid = cid * NS + sid

    pltpu.sync_copy(s_hbm, s_v)
    pltpu.sync_copy(d_hbm, d_v)
    pltpu.sync_copy(src_hbm.at[wid], srcb)
    pltpu.sync_copy(dst_hbm.at[wid], dstb)
    pltpu.sync_copy(el_hbm.at[pl.ds(wid * EPW, EPW)], elb)

    def zrow(r, carry):
        for cc in range(ROW // 16):
            zb[r, pl.ds(cc * 16, 16)] = jnp.zeros((16,), jnp.float32)
        return carry

    lax.fori_loop(0, ZROWS, zrow, 0)
    for t in range(NPT // ZROWS):
        pltpu.sync_copy(zb, acc.at[pl.ds(sid * NPT + t * ZROWS, ZROWS)])
    plsc.subcore_barrier()

    iota16 = lax.iota(jnp.int32, 16)

    def phase(b, carry):
        cp = pltpu.async_copy(xl_hbm.at[srcb.at[b]], rows, gsem)
        for gi in range(K // 16):
            srci = srcb[b, pl.ds(gi * 16, 16)]
            dsti = dstb[b, pl.ds(gi * 16, 16)]
            sv = plsc.load_gather(s_v, [srci])
            dv = plsc.load_gather(d_v, [dsti])
            a = sv + dv + elb[pl.ds(b * K + gi * 16, 16)]
            a = jnp.where(a >= 0, a, 0.2 * a)
            exb[pl.ds(gi * 16, 16)] = jnp.exp(a)
        cp.wait()

        def row_body(j, rcarry):
            exj = plsc.load_gather(exb, [jnp.full((16,), j, jnp.int32)])
            for cc in range(C // 16):
                srows[j, pl.ds(cc * 16, 16)] = rows[j, pl.ds(cc * 16, 16)] * exj
            srows[j, pl.ds(C, 16)] = jnp.where(iota16 == 0, exj, 0.0)
            return rcarry

        lax.fori_loop(0, K, row_body, 0)
        pltpu.sync_copy(srows, acc.at[dstb.at[b]], add=True)
        return carry

    lax.fori_loop(0, NB, phase, 0)

    plsc.subcore_barrier()
    for t in range(NPT // ZROWS):
        off = sid * NPT + t * ZROWS
        pltpu.sync_copy(acc.at[pl.ds(off, ZROWS)],
                        out_hbm.at[pl.ds(cid * NPAD + off, ZROWS)])


@functools.cache
def _make_sc_edge_pass():
    return pl.kernel(
        _sc_body,
        out_type=jax.ShapeDtypeStruct((NC * NPAD, ROW), jnp.float32),
        compiler_params=pltpu.CompilerParams(
            needs_layout_passes=False, use_tc_tiling_on_sc=False),
        mesh=plsc.VectorSubcoreMesh(
            core_axis_name="c", subcore_axis_name="s",
            num_cores=NC, num_subcores=NS),
        scratch_types=[
            pltpu.VMEM((N,), jnp.float32),
            pltpu.VMEM((N,), jnp.float32),
            pltpu.VMEM((NB, K), jnp.int32),
            pltpu.VMEM((NB, K), jnp.int32),
            pltpu.VMEM((EPW,), jnp.float32),
            pltpu.VMEM((K, C), jnp.float32),
            pltpu.VMEM((K, ROW), jnp.float32),
            pltpu.VMEM((K,), jnp.float32),
            pltpu.VMEM((ZROWS, ROW), jnp.float32),
            pltpu.MemorySpace.VMEM_SHARED((NPAD, ROW), jnp.float32),
            pltpu.SemaphoreType.DMA,
        ],
    )


def kernel(x, edge_index, edge_attr, batch, W1, a_src1, a_dst1, We1, a_e1, b1,
           W2, a_src2, a_dst2, We2, a_e2, b2):
    src3 = edge_index[0].reshape(NW, NB, K)
    dst3 = edge_index[1].reshape(NW, NB, K)
    batch_row = batch.reshape(1, N)

    ea_sum = _ea_colsum(edge_attr)

    avec1 = jnp.stack([a_src1, a_dst1], axis=1)
    avec2 = jnp.stack([a_src2, a_dst2], axis=1)

    el1 = _edge_logits(edge_attr, We1, a_e1.reshape(C, 1)).reshape(E)
    el2 = _edge_logits(edge_attr, We2, a_e2.reshape(C, 1)).reshape(E)

    sc_pass = _make_sc_edge_pass()

    xl1, sd1, aself1 = _prep(x, W1, avec1, We1, a_e1.reshape(C, 1), ea_sum)
    acc1 = sc_pass(xl1, sd1[:, 0], sd1[:, 1], src3, dst3, el1)
    h1 = _combine(acc1, aself1, xl1, b1.reshape(1, C))

    xl2, sd2, aself2 = _prep(h1, W2, avec2, We2, a_e2.reshape(C, 1), ea_sum)
    acc2 = sc_pass(xl2, sd2[:, 0], sd2[:, 1], src3, dst3, el2)
    return _combine_pool(acc2, aself2, xl2, b2.reshape(1, C), batch_row)


# speedup vs baseline: 1.1314x; 1.0435x over previous
import functools

import jax
import jax.numpy as jnp
from jax import lax
from jax.experimental import pallas as pl
from jax.experimental.pallas import tpu as pltpu
from jax.experimental.pallas import tpu_sc as plsc

N = 10000
E = 320000
D_IN = 128
D_EDGE = 16
C = 64
G = 64

NC = 2
NS = 16
NW = NC * NS
EPW = E // NW
K = 80
NB = EPW // K
NPAD = 10240
NPT = NPAD // NS
ZROWS = 32
ROW = 80

EBLK = E // 16


def _ea_sum_body(ea_ref, out_ref):
    i = pl.program_id(0)

    @pl.when(i == 0)
    def _():
        out_ref[...] = jnp.zeros_like(out_ref)

    out_ref[...] += jnp.sum(ea_ref[...], axis=0, keepdims=True)


def _ea_colsum(edge_attr):
    return pl.pallas_call(
        _ea_sum_body,
        grid=(16,),
        in_specs=[pl.BlockSpec((EBLK, D_EDGE), lambda i: (i, 0))],
        out_specs=pl.BlockSpec((1, D_EDGE), lambda i: (0, 0)),
        out_shape=jax.ShapeDtypeStruct((1, D_EDGE), jnp.float32),
    )(edge_attr)


def _el_body(ea_ref, we_ref, ae_ref, el_ref):
    t = jnp.dot(ea_ref[...], we_ref[...], preferred_element_type=jnp.float32)
    el_ref[...] = jnp.dot(t, ae_ref[...], preferred_element_type=jnp.float32)


def _edge_logits(edge_attr, We, ae):
    return pl.pallas_call(
        _el_body,
        grid=(16,),
        in_specs=[
            pl.BlockSpec((EBLK, D_EDGE), lambda i: (i, 0)),
            pl.BlockSpec((D_EDGE, C), lambda i: (0, 0)),
            pl.BlockSpec((C, 1), lambda i: (0, 0)),
        ],
        out_specs=pl.BlockSpec((EBLK, 1), lambda i: (i, 0)),
        out_shape=jax.ShapeDtypeStruct((E, 1), jnp.float32),
    )(edge_attr, We, ae)


def _prep_body(x_ref, w_ref, av_ref, we_ref, ae_ref, es_ref,
               xl_ref, sd_ref, as_ref):
    xl = jnp.dot(x_ref[...], w_ref[...], preferred_element_type=jnp.float32)
    xl_ref[...] = xl
    sd = jnp.dot(xl, av_ref[...], preferred_element_type=jnp.float32)
    sd_ref[...] = sd
    t = jnp.dot(es_ref[...] * (1.0 / E), we_ref[...],
                preferred_element_type=jnp.float32)
    c0 = jnp.dot(t, ae_ref[...], preferred_element_type=jnp.float32)
    a = sd[:, 0:1] + sd[:, 1:2] + c0
    a = jnp.where(a >= 0, a, 0.2 * a)
    as_ref[...] = jnp.exp(a)


def _prep(xin, W, avec, We, ae, ea_sum):
    n, _ = xin.shape
    return pl.pallas_call(
        _prep_body,
        out_shape=(
            jax.ShapeDtypeStruct((n, C), jnp.float32),
            jax.ShapeDtypeStruct((n, 2), jnp.float32),
            jax.ShapeDtypeStruct((n, 1), jnp.float32),
        ),
    )(xin, W, avec, We, ae, ea_sum)


def _comb_body(acc_ref, as_ref, xl_ref, b_ref, h_ref):
    ns = acc_ref[0:N, :] + acc_ref[NPAD:NPAD + N, :]
    num = ns[:, 0:C] + as_ref[...] * xl_ref[...]
    den = ns[:, C:C + 1] + as_ref[...] + 1e-16
    h_ref[...] = num / den + b_ref[...]


def _combine(acc, aself, xl, brow):
    return pl.pallas_call(
        _comb_body,
        out_shape=jax.ShapeDtypeStruct((N, C), jnp.float32),
    )(acc, aself, xl, brow)


def _pool_body(acc_ref, as_ref, xl_ref, b_ref, batch_ref, out_ref):
    ns = acc_ref[0:N, :] + acc_ref[NPAD:NPAD + N, :]
    num = ns[:, 0:C] + as_ref[...] * xl_ref[...]
    den = ns[:, C:C + 1] + as_ref[...] + 1e-16
    h = num / den + b_ref[...]
    gids = lax.broadcasted_iota(jnp.int32, (G, N), 0)
    p = (gids == batch_ref[...]).astype(jnp.float32)
    out_ref[...] = jnp.dot(p, h, preferred_element_type=jnp.float32,
                           precision=lax.Precision.HIGHEST)


def _combine_pool(acc, aself, xl, brow, batch_row):
    return pl.pallas_call(
        _pool_body,
        out_shape=jax.ShapeDtypeStruct((G, C), jnp.float32),
    )(acc, aself, xl, brow, batch_row)


def _sc_body(xl_hbm, s_hbm, d_hbm, src_hbm, dst_hbm, el_hbm, out_hbm,
             s_v, d_v, srcb, dstb, elb, rows0, rows1, srows0, srows1,
             exb, zb, acc, gsem0, gsem1, ssem0, ssem1):
    cid = lax.axis_index("c")
    sid = lax.axis_index("s")
    wid = cid * NS + sid
    rowss = (rows0, rows1)
    srowss = (srows0, srows1)
    gsems = (gsem0, gsem1)
    ssems = (ssem0, ssem1)

    pltpu.sync_copy(s_hbm, s_v)
    pltpu.sync_copy(d_hbm, d_v)
    pltpu.sync_copy(src_hbm.at[wid], srcb)
    pltpu.sync_copy(dst_hbm.at[wid], dstb)
    pltpu.sync_copy(el_hbm.at[pl.ds(wid * EPW, EPW)], elb)

    def zrow(r, carry):
        for cc in range(ROW // 16):
            zb[r, pl.ds(cc * 16, 16)] = jnp.zeros((16,), jnp.float32)
        return carry

    lax.fori_loop(0, ZROWS, zrow, 0)
    for t in range(NPT // ZROWS):
        pltpu.sync_copy(zb, acc.at[pl.ds(sid * NPT + t * ZROWS, ZROWS)])
    plsc.subcore_barrier()

    iota16 = lax.iota(jnp.int32, 16)

    def wait_scatter(b, p):
        pltpu.make_async_copy(srowss[p], acc.at[dstb.at[b]],
                              ssems[p]).wait()

    def phase(b, p):
        rows = rowss[p]
        srows = srowss[p]
        cp = pltpu.async_copy(xl_hbm.at[srcb.at[b]], rows, gsems[p])
        for gi in range(K // 16):
            srci = srcb[b, pl.ds(gi * 16, 16)]
            dsti = dstb[b, pl.ds(gi * 16, 16)]
            sv = plsc.load_gather(s_v, [srci])
            dv = plsc.load_gather(d_v, [dsti])
            a = sv + dv + elb[pl.ds(b * K + gi * 16, 16)]
            a = jnp.where(a >= 0, a, 0.2 * a)
            exb[pl.ds(gi * 16, 16)] = jnp.exp(a)
        @pl.when(b >= 2)
        def _():
            wait_scatter(b - 2, p)
        cp.wait()

        def row_body(j, rcarry):
            exj = plsc.load_gather(exb, [jnp.full((16,), j, jnp.int32)])
            for cc in range(C // 16):
                srows[j, pl.ds(cc * 16, 16)] = rows[j, pl.ds(cc * 16, 16)] * exj
            srows[j, pl.ds(C, 16)] = jnp.where(iota16 == 0, exj, 0.0)
            return rcarry

        lax.fori_loop(0, K, row_body, 0)
        pltpu.async_copy(srows, acc.at[dstb.at[b]], ssems[p], add=True)

    def super_body(s, carry):
        phase(2 * s, 0)
        phase(2 * s + 1, 1)
        return carry

    lax.fori_loop(0, (NB - 1) // 2, super_body, 0)
    phase(NB - 1, 0)
    wait_scatter(NB - 2, 1)
    wait_scatter(NB - 1, 0)

    plsc.subcore_barrier()
    for t in range(NPT // ZROWS):
        off = sid * NPT + t * ZROWS
        pltpu.sync_copy(acc.at[pl.ds(off, ZROWS)],
                        out_hbm.at[pl.ds(cid * NPAD + off, ZROWS)])


@functools.cache
def _make_sc_edge_pass():
    return pl.kernel(
        _sc_body,
        out_type=jax.ShapeDtypeStruct((NC * NPAD, ROW), jnp.float32),
        compiler_params=pltpu.CompilerParams(
            needs_layout_passes=False, use_tc_tiling_on_sc=False),
        mesh=plsc.VectorSubcoreMesh(
            core_axis_name="c", subcore_axis_name="s",
            num_cores=NC, num_subcores=NS),
        scratch_types=[
            pltpu.VMEM((N,), jnp.float32),
            pltpu.VMEM((N,), jnp.float32),
            pltpu.VMEM((NB, K), jnp.int32),
            pltpu.VMEM((NB, K), jnp.int32),
            pltpu.VMEM((EPW,), jnp.float32),
            pltpu.VMEM((K, C), jnp.float32),
            pltpu.VMEM((K, C), jnp.float32),
            pltpu.VMEM((K, ROW), jnp.float32),
            pltpu.VMEM((K, ROW), jnp.float32),
            pltpu.VMEM((K,), jnp.float32),
            pltpu.VMEM((ZROWS, ROW), jnp.float32),
            pltpu.MemorySpace.VMEM_SHARED((NPAD, ROW), jnp.float32),
            pltpu.SemaphoreType.DMA,
            pltpu.SemaphoreType.DMA,
            pltpu.SemaphoreType.DMA,
            pltpu.SemaphoreType.DMA,
        ],
    )


def kernel(x, edge_index, edge_attr, batch, W1, a_src1, a_dst1, We1, a_e1, b1,
           W2, a_src2, a_dst2, We2, a_e2, b2):
    src3 = edge_index[0].reshape(NW, NB, K)
    dst3 = edge_index[1].reshape(NW, NB, K)
    batch_row = batch.reshape(1, N)

    ea_sum = _ea_colsum(edge_attr)

    avec1 = jnp.stack([a_src1, a_dst1], axis=1)
    avec2 = jnp.stack([a_src2, a_dst2], axis=1)

    el1 = _edge_logits(edge_attr, We1, a_e1.reshape(C, 1)).reshape(E)
    el2 = _edge_logits(edge_attr, We2, a_e2.reshape(C, 1)).reshape(E)

    sc_pass = _make_sc_edge_pass()

    xl1, sd1, aself1 = _prep(x, W1, avec1, We1, a_e1.reshape(C, 1), ea_sum)
    acc1 = sc_pass(xl1, sd1[:, 0], sd1[:, 1], src3, dst3, el1)
    h1 = _combine(acc1, aself1, xl1, b1.reshape(1, C))

    xl2, sd2, aself2 = _prep(h1, W2, avec2, We2, a_e2.reshape(C, 1), ea_sum)
    acc2 = sc_pass(xl2, sd2[:, 0], sd2[:, 1], src3, dst3, el2)
    return _combine_pool(acc2, aself2, xl2, b2.reshape(1, C), batch_row)
